# Initial kernel scaffold; baseline (speedup 1.0000x reference)
import jax
import jax.numpy as jnp
from jax.experimental import pallas as pl
from jax.experimental.pallas import tpu as pltpu

D_MODEL = 1024
D_FF = 2048
N_EXP = 8
T = 2048
TB = 1024
NEG = -1e30


def _moe_dense_kernel(x32_ref, xb_ref, w1_ref, b1_ref, w2_ref, b2_ref,
                      wg_ref, bg_ref, out_ref, acc_ref, w_ref):
    e = pl.program_id(1)

    @pl.when(e == 0)
    def _gating():
        g = jnp.dot(x32_ref[...], wg_ref[...],
                    preferred_element_type=jnp.float32,
                    precision=jax.lax.Precision.HIGHEST) + bg_ref[0, :]
        idx = jax.lax.broadcasted_iota(jnp.int32, g.shape, 1)
        m0 = jnp.max(g, axis=1, keepdims=True)
        e0 = jnp.min(jnp.where(g >= m0, idx, N_EXP), axis=1, keepdims=True)
        gm = jnp.where(idx == e0, NEG, g)
        m1 = jnp.max(gm, axis=1, keepdims=True)
        e1 = jnp.min(jnp.where(gm >= m1, idx, N_EXP), axis=1, keepdims=True)
        p0 = 1.0 / (1.0 + jnp.exp(m1 - m0))
        p1 = 1.0 - p0
        w_ref[...] = jnp.where(idx == e0, p0, 0.0) + jnp.where(idx == e1, p1, 0.0)

    xb = xb_ref[...]
    h = jnp.dot(xb, w1_ref[0], preferred_element_type=jnp.float32)
    h = jax.nn.gelu(h + b1_ref[0, :], approximate=False)
    y = jnp.dot(h.astype(jnp.bfloat16), w2_ref[0],
                preferred_element_type=jnp.float32) + b2_ref[0, :]
    contrib = w_ref[:, e][:, None] * y

    @pl.when(e == 0)
    def _init():
        acc_ref[...] = contrib

    @pl.when(e > 0)
    def _accum():
        acc_ref[...] += contrib

    @pl.when(e == N_EXP - 1)
    def _flush():
        out_ref[...] = acc_ref[...]


def kernel(x, W1, b1, W2, b2, Wg, bg):
    B, S, D = x.shape
    x2d = x.reshape(-1, D)
    xb = x2d.astype(jnp.bfloat16)
    w1b = W1.astype(jnp.bfloat16)
    w2b = W2.astype(jnp.bfloat16)
    bg2 = bg.reshape(1, N_EXP)

    out = pl.pallas_call(
        _moe_dense_kernel,
        grid=(T // TB, N_EXP),
        in_specs=[
            pl.BlockSpec((TB, D_MODEL), lambda t, e: (t, 0)),
            pl.BlockSpec((TB, D_MODEL), lambda t, e: (t, 0)),
            pl.BlockSpec((1, D_MODEL, D_FF), lambda t, e: (e, 0, 0)),
            pl.BlockSpec((1, D_FF), lambda t, e: (e, 0)),
            pl.BlockSpec((1, D_FF, D_MODEL), lambda t, e: (e, 0, 0)),
            pl.BlockSpec((1, D_MODEL), lambda t, e: (e, 0)),
            pl.BlockSpec((D_MODEL, N_EXP), lambda t, e: (0, 0)),
            pl.BlockSpec((1, N_EXP), lambda t, e: (0, 0)),
        ],
        out_specs=pl.BlockSpec((TB, D_MODEL), lambda t, e: (t, 0)),
        out_shape=jax.ShapeDtypeStruct((T, D_MODEL), jnp.float32),
        scratch_shapes=[
            pltpu.VMEM((TB, D_MODEL), jnp.float32),
            pltpu.VMEM((TB, N_EXP), jnp.float32),
        ],
    )(x2d, xb, w1b, b1, w2b, b2, Wg, bg2)
    return out.reshape(B, S, D)


# speedup vs baseline: 2.2187x; 2.2187x over previous
import jax
import jax.numpy as jnp
from jax.experimental import pallas as pl
from jax.experimental.pallas import tpu as pltpu

D_MODEL = 1024
D_FF = 2048
N_EXP = 8
T = 2048
TB = 1024
NEG = -1e30


def _moe_dense_kernel(x32_ref, xb_ref, w1_ref, b1_ref, w2_ref, b2_ref,
                      wg_ref, bg_ref, out_ref, acc_ref, w_ref):
    e = pl.program_id(1)

    @pl.when(e == 0)
    def _gating():
        g = jnp.dot(x32_ref[...], wg_ref[...],
                    preferred_element_type=jnp.float32) + bg_ref[0, :]
        idx = jax.lax.broadcasted_iota(jnp.int32, g.shape, 1)
        m0 = jnp.max(g, axis=1, keepdims=True)
        e0 = jnp.min(jnp.where(g >= m0, idx, N_EXP), axis=1, keepdims=True)
        gm = jnp.where(idx == e0, NEG, g)
        m1 = jnp.max(gm, axis=1, keepdims=True)
        e1 = jnp.min(jnp.where(gm >= m1, idx, N_EXP), axis=1, keepdims=True)
        p0 = 1.0 / (1.0 + jnp.exp(m1 - m0))
        p1 = 1.0 - p0
        w_ref[...] = jnp.where(idx == e0, p0, 0.0) + jnp.where(idx == e1, p1, 0.0)

    xb = xb_ref[...]
    h = jnp.dot(xb, w1_ref[0], preferred_element_type=jnp.float32)
    h = h + b1_ref[0, 0, :]
    h = 0.5 * h * (1.0 + jax.lax.erf(h * 0.7071067811865476))
    y = jnp.dot(h.astype(jnp.bfloat16), w2_ref[0],
                preferred_element_type=jnp.float32) + b2_ref[0, 0, :]
    wfull = w_ref[...]
    lane = jax.lax.broadcasted_iota(jnp.int32, wfull.shape, 1)
    wcol = jnp.sum(jnp.where(lane == e, wfull, 0.0), axis=1, keepdims=True)
    contrib = wcol * y

    @pl.when(e == 0)
    def _init():
        acc_ref[...] = contrib

    @pl.when(e > 0)
    def _accum():
        acc_ref[...] += contrib

    @pl.when(e == N_EXP - 1)
    def _flush():
        out_ref[...] = acc_ref[...]


def kernel(x, W1, b1, W2, b2, Wg, bg):
    B, S, D = x.shape
    x2d = x.reshape(-1, D)
    xb = x2d.astype(jnp.bfloat16)
    w1b = W1.astype(jnp.bfloat16)
    w2b = W2.astype(jnp.bfloat16)
    bg2 = bg.reshape(1, N_EXP)

    out = pl.pallas_call(
        _moe_dense_kernel,
        grid=(T // TB, N_EXP),
        in_specs=[
            pl.BlockSpec((TB, D_MODEL), lambda t, e: (t, 0)),
            pl.BlockSpec((TB, D_MODEL), lambda t, e: (t, 0)),
            pl.BlockSpec((1, D_MODEL, D_FF), lambda t, e: (e, 0, 0)),
            pl.BlockSpec((1, 1, D_FF), lambda t, e: (e, 0, 0)),
            pl.BlockSpec((1, D_FF, D_MODEL), lambda t, e: (e, 0, 0)),
            pl.BlockSpec((1, 1, D_MODEL), lambda t, e: (e, 0, 0)),
            pl.BlockSpec((D_MODEL, N_EXP), lambda t, e: (0, 0)),
            pl.BlockSpec((1, N_EXP), lambda t, e: (0, 0)),
        ],
        out_specs=pl.BlockSpec((TB, D_MODEL), lambda t, e: (t, 0)),
        out_shape=jax.ShapeDtypeStruct((T, D_MODEL), jnp.float32),
        scratch_shapes=[
            pltpu.VMEM((TB, D_MODEL), jnp.float32),
            pltpu.VMEM((TB, N_EXP), jnp.float32),
        ],
    )(x2d, xb, w1b, b1.reshape(N_EXP, 1, D_FF), w2b,
      b2.reshape(N_EXP, 1, D_MODEL), Wg, bg2)
    return out.reshape(B, S, D)


# speedup vs baseline: 2.4790x; 1.1173x over previous
import functools

import jax
import jax.numpy as jnp
from jax import lax
from jax.experimental import pallas as pl
from jax.experimental.pallas import tpu as pltpu
from jax.experimental.pallas import tpu_sc as plsc

D_MODEL = 1024
D_FF = 2048
N_EXP = 8
T = 2048
K = 2
NS = 4096
BLK = 256
NB = 24
NSORT = NB * BLK
NEG = -1e30

NC = 2
NSUB = 16
NW = NC * NSUB
ROWS_B = NS // NW
ROWS_D = T // NW
CHUNK = 32


def _cumsum_rows(a):
    n = a.shape[0]
    s = 1
    while s < n:
        sh = jnp.concatenate(
            [jnp.zeros((s, a.shape[1]), a.dtype), a[:n - s]], axis=0)
        a = a + sh
        s *= 2
    return a


def _cumsum_lanes(a):
    n = a.shape[1]
    s = 1
    while s < n:
        sh = jnp.concatenate(
            [jnp.zeros((a.shape[0], s), a.dtype), a[:, :n - s]], axis=1)
        a = a + sh
        s *= 2
    return a


def _gating_kernel(x_ref, wg_ref, bg_ref, pos_ref, prob_ref, beidx_ref,
                   beact_ref):
    g = jnp.dot(x_ref[...], wg_ref[...],
                preferred_element_type=jnp.float32) + bg_ref[0, :]
    idx8 = lax.broadcasted_iota(jnp.int32, (T, N_EXP), 1)
    m0 = jnp.max(g, axis=1, keepdims=True)
    e0 = jnp.min(jnp.where(g >= m0, idx8, N_EXP), axis=1, keepdims=True)
    gm = jnp.where(idx8 == e0, NEG, g)
    m1 = jnp.max(gm, axis=1, keepdims=True)
    e1 = jnp.min(jnp.where(gm >= m1, idx8, N_EXP), axis=1, keepdims=True)
    p0 = 1.0 / (1.0 + jnp.exp(m1 - m0))
    p1 = 1.0 - p0

    oh0 = (idx8 == e0).astype(jnp.float32)
    oh1 = (idx8 == e1).astype(jnp.float32)
    c01 = _cumsum_rows(jnp.concatenate([oh0, oh1], axis=1))
    c0 = c01[:, :N_EXP]
    c1 = c01[:, N_EXP:]
    cnt0 = c0[T - 1:T, :]
    cnt = cnt0 + c1[T - 1:T, :]
    padded = jnp.ceil(cnt * (1.0 / BLK)) * BLK
    off = _cumsum_lanes(padded) - padded

    r0 = jnp.sum(oh0 * (off + c0), axis=1, keepdims=True) - 1.0
    r1 = jnp.sum(oh1 * (off + cnt0 + c1), axis=1, keepdims=True) - 1.0
    pos_ref[0:T, :] = r0.astype(jnp.int32)
    pos_ref[T:NS, :] = r1.astype(jnp.int32)

    prob_ref[0:T, :] = jnp.broadcast_to(p0, (T, 128))
    prob_ref[T:NS, :] = jnp.broadcast_to(p1, (T, 128))

    ends = off + padded
    total = jnp.sum(padded, axis=1, keepdims=True)
    nbv = lax.broadcasted_iota(jnp.int32, (NB, 1), 0).astype(jnp.float32) * BLK
    be = jnp.sum((nbv >= ends).astype(jnp.float32), axis=1, keepdims=True)
    act = (nbv < total)
    beidx_ref[...] = jnp.minimum(be, N_EXP - 1).astype(jnp.int32)
    beact_ref[...] = act.astype(jnp.int32)


def _gating(x2d, Wg, bg2):
    return pl.pallas_call(
        _gating_kernel,
        grid=(1,),
        in_specs=[
            pl.BlockSpec((T, D_MODEL), lambda i: (0, 0)),
            pl.BlockSpec((D_MODEL, N_EXP), lambda i: (0, 0)),
            pl.BlockSpec((1, N_EXP), lambda i: (0, 0)),
        ],
        out_specs=[
            pl.BlockSpec((NS, 1), lambda i: (0, 0)),
            pl.BlockSpec((NS, 128), lambda i: (0, 0)),
            pl.BlockSpec((NB, 1), lambda i: (0, 0)),
            pl.BlockSpec((NB, 1), lambda i: (0, 0)),
        ],
        out_shape=[
            jax.ShapeDtypeStruct((NS, 1), jnp.int32),
            jax.ShapeDtypeStruct((NS, 128), jnp.float32),
            jax.ShapeDtypeStruct((NB, 1), jnp.int32),
            jax.ShapeDtypeStruct((NB, 1), jnp.int32),
        ],
    )(x2d, Wg, bg2)


@functools.lru_cache(maxsize=None)
def _make_scatter():
    mesh = plsc.VectorSubcoreMesh(core_axis_name="c", subcore_axis_name="s")

    @functools.partial(
        pl.kernel,
        mesh=mesh,
        out_type=[
            jax.ShapeDtypeStruct((NSORT, D_MODEL), jnp.float32),
            jax.ShapeDtypeStruct((NSORT, 128), jnp.float32),
        ],
        scratch_types=[
            pltpu.VMEM((CHUNK,), jnp.int32),
            pltpu.VMEM((CHUNK, D_MODEL), jnp.float32),
            pltpu.VMEM((CHUNK, 128), jnp.float32),
        ],
    )
    def scatter_k(x_hbm, pos_hbm, prob_hbm, xs_hbm, ps_hbm, idx_v, row_v,
                  prb_v):
        wid = lax.axis_index("s") * NC + lax.axis_index("c")
        slot_base = wid * ROWS_B
        tok_base = (wid % NSUB) * ROWS_B

        @pl.loop(0, ROWS_B // CHUNK)
        def _(c):
            sb = slot_base + c * CHUNK
            tb = tok_base + c * CHUNK
            pltpu.sync_copy(pos_hbm.at[pl.ds(sb, CHUNK)], idx_v)
            pltpu.sync_copy(x_hbm.at[pl.ds(tb, CHUNK)], row_v)
            pltpu.sync_copy(prob_hbm.at[pl.ds(sb, CHUNK)], prb_v)
            pltpu.sync_copy(row_v, xs_hbm.at[idx_v])
            pltpu.sync_copy(prb_v, ps_hbm.at[idx_v])

    return scatter_k


def _ffn_kernel(beidx_ref, beact_ref, x_ref, w1_ref, b1_ref, w2_ref, b2_ref,
                prob_ref, y_ref):
    nb = pl.program_id(0)

    @pl.when(beact_ref[nb] > 0)
    def _():
        xb = x_ref[...].astype(jnp.bfloat16)
        h = jnp.dot(xb, w1_ref[0], preferred_element_type=jnp.float32)
        h = h + b1_ref[0, 0, :]
        h = 0.5 * h * (1.0 + lax.erf(h * 0.7071067811865476))
        y = jnp.dot(h.astype(jnp.bfloat16), w2_ref[0],
                    preferred_element_type=jnp.float32) + b2_ref[0, 0, :]
        y_ref[...] = y * prob_ref[:, 0:1]


def _ffn(beidx, beact, xs, w1b, b1r, w2b, b2r, ps):
    grid_spec = pltpu.PrefetchScalarGridSpec(
        num_scalar_prefetch=2,
        grid=(NB,),
        in_specs=[
            pl.BlockSpec((BLK, D_MODEL), lambda nb, bi, ba: (nb, 0)),
            pl.BlockSpec((1, D_MODEL, D_FF), lambda nb, bi, ba: (bi[nb], 0, 0)),
            pl.BlockSpec((1, 1, D_FF), lambda nb, bi, ba: (bi[nb], 0, 0)),
            pl.BlockSpec((1, D_FF, D_MODEL), lambda nb, bi, ba: (bi[nb], 0, 0)),
            pl.BlockSpec((1, 1, D_MODEL), lambda nb, bi, ba: (bi[nb], 0, 0)),
            pl.BlockSpec((BLK, 128), lambda nb, bi, ba: (nb, 0)),
        ],
        out_specs=pl.BlockSpec((BLK, D_MODEL), lambda nb, bi, ba: (nb, 0)),
    )
    return pl.pallas_call(
        _ffn_kernel,
        grid_spec=grid_spec,
        out_shape=jax.ShapeDtypeStruct((NSORT, D_MODEL), jnp.float32),
    )(beidx, beact, xs, w1b, b1r, w2b, b2r, ps)


@functools.lru_cache(maxsize=None)
def _make_combine():
    mesh = plsc.VectorSubcoreMesh(core_axis_name="c", subcore_axis_name="s")

    @functools.partial(
        pl.kernel,
        mesh=mesh,
        out_type=jax.ShapeDtypeStruct((T, D_MODEL), jnp.float32),
        scratch_types=[
            pltpu.VMEM((CHUNK,), jnp.int32),
            pltpu.VMEM((CHUNK,), jnp.int32),
            pltpu.VMEM((CHUNK, D_MODEL), jnp.float32),
            pltpu.VMEM((CHUNK, D_MODEL), jnp.float32),
            pltpu.SemaphoreType.DMA,
        ],
    )
    def combine_k(y_hbm, pos_hbm, o_hbm, i0_v, i1_v, g0_v, g1_v, sem):
        wid = lax.axis_index("s") * NC + lax.axis_index("c")
        tok_base = wid * ROWS_D

        @pl.loop(0, ROWS_D // CHUNK)
        def _(c):
            tb = tok_base + c * CHUNK
            pltpu.sync_copy(pos_hbm.at[pl.ds(tb, CHUNK)], i0_v)
            pltpu.sync_copy(pos_hbm.at[pl.ds(T + tb, CHUNK)], i1_v)
            cp0 = pltpu.async_copy(y_hbm.at[i0_v], g0_v, sem)
            cp1 = pltpu.async_copy(y_hbm.at[i1_v], g1_v, sem)
            cp0.wait()
            cp1.wait()

            @pl.loop(0, CHUNK)
            def _(r):
                @pl.loop(0, D_MODEL, step=16)
                def _(cc):
                    sl = (pl.ds(r, 1), pl.ds(cc, 16))
                    g0_v.at[*sl][...] = g0_v.at[*sl][...] + g1_v.at[*sl][...]

            pltpu.sync_copy(g0_v, o_hbm.at[pl.ds(tb, CHUNK)])

    return combine_k


def kernel(x, W1, b1, W2, b2, Wg, bg):
    B, S, D = x.shape
    x2d = x.reshape(T, D)
    w1b = W1.astype(jnp.bfloat16)
    w2b = W2.astype(jnp.bfloat16)
    b1r = b1.reshape(N_EXP, 1, D_FF)
    b2r = b2.reshape(N_EXP, 1, D_MODEL)
    bg2 = bg.reshape(1, N_EXP)

    pos, prob, beidx, beact = _gating(x2d, Wg, bg2)
    pos1d = pos.reshape(NS)
    xs, ps = _make_scatter()(x2d, pos1d, prob)
    ys = _ffn(beidx.reshape(NB), beact.reshape(NB), xs, w1b, b1r, w2b, b2r, ps)
    out = _make_combine()(ys, pos1d)
    return out.reshape(B, S, D)
